# baseline (speedup 1.0000x reference)
import functools

import jax
import jax.numpy as jnp
from jax import lax
from jax.experimental import pallas as pl
from jax.experimental.pallas import tpu as pltpu
from jax.experimental.pallas import tpu_sc as plsc

MAX_LEN = 1000
HIDDEN_DIM = 128

NW = 32
B, N, L, D = 16, 64, 50, 128
PANELS = B * L
PANELS_PER_W = PANELS // NW
L_PER_W = L // 2
CH_SIZES = [4] * 6 + [1]
CH_OFFS = [sum(CH_SIZES[:i]) for i in range(len(CH_SIZES))]
NCH = len(CH_SIZES)
CH_MAX = max(CH_SIZES)
NBUF = 3
GA_SKEW = 1
OUT_SKEW = 2


def _sc_kernel(emb_hbm, pos_hbm, pe_hbm, out_hbm,
               idx_v, pe_sh, b0, b1, b2,
               sem_in, sem_ga, sem_out, sem_idx):
  bufs = (b0, b1, b2)
  sid = lax.axis_index("s")
  wid = sid * 2 + lax.axis_index("c")
  bb = wid // 2
  l0 = (wid % 2) * L_PER_W

  in_d = [None] * NCH
  ga_d = [None] * NCH
  out_d = [None] * NCH

  def start_in(c):
    sz = CH_SIZES[c]
    in_d[c] = pltpu.async_copy(
        emb_hbm.at[bb, pl.ds(l0 + CH_OFFS[c], sz)],
        bufs[c % NBUF].at[pl.ds(0, sz)], sem_in)

  def start_ga(c):
    in_d[c].wait()
    ga_d[c] = [
        pltpu.async_copy(
            pe_sh.at[idx_v.at[CH_OFFS[c] + k]], bufs[c % NBUF].at[k],
            sem_ga, add=True)
        for k in range(CH_SIZES[c])
    ]

  def start_out(c):
    for d in ga_d[c]:
      d.wait()
    sz = CH_SIZES[c]
    out_d[c] = pltpu.async_copy(
        bufs[c % NBUF].at[pl.ds(0, sz)],
        out_hbm.at[bb, pl.ds(l0 + CH_OFFS[c], sz)], sem_out)

  idx_cp = pltpu.async_copy(
      pos_hbm.at[pl.ds(l0, L_PER_W), bb], idx_v, sem_idx)
  for c in range(GA_SKEW):
    start_in(c)

  @pl.when(sid == 0)
  def _stage():
    pltpu.sync_copy(pe_hbm, pe_sh)

  idx_cp.wait()
  plsc.subcore_barrier()

  for c in range(NCH):
    if c >= NBUF:
      out_d[c - NBUF].wait()
    if c >= GA_SKEW:
      start_in(c)
      start_ga(c - GA_SKEW)
    if c >= OUT_SKEW:
      start_out(c - OUT_SKEW)

  for c in range(NCH - GA_SKEW, NCH):
    start_ga(c)
  for c in range(NCH - OUT_SKEW, NCH):
    start_out(c)
  for c in range(max(0, NCH - NBUF), NCH):
    out_d[c].wait()


def kernel(input_emb, position, pe):
  emb_t = jnp.transpose(input_emb, (0, 2, 1, 3))
  pos_t = jnp.transpose(position.astype(jnp.int32), (2, 0, 1))

  run = functools.partial(
      pl.kernel,
      mesh=plsc.VectorSubcoreMesh(core_axis_name="c", subcore_axis_name="s"),
      out_type=jax.ShapeDtypeStruct((B, L, N, D), jnp.float32),
      scratch_types=[
          pltpu.VMEM((PANELS_PER_W, N), jnp.int32),
          pltpu.VMEM_SHARED((MAX_LEN, D), jnp.float32),
          pltpu.VMEM((CH_MAX, N, D), jnp.float32),
          pltpu.VMEM((CH_MAX, N, D), jnp.float32),
          pltpu.VMEM((CH_MAX, N, D), jnp.float32),
          pltpu.SemaphoreType.DMA,
          pltpu.SemaphoreType.DMA,
          pltpu.SemaphoreType.DMA,
          pltpu.SemaphoreType.DMA,
      ],
  )(_sc_kernel)

  out_t = run(emb_t, pos_t, pe)
  return jnp.transpose(out_t, (0, 2, 1, 3))


# speedup vs baseline: 1.0145x; 1.0145x over previous
import functools

import jax
import jax.numpy as jnp
from jax import lax
from jax.experimental import pallas as pl
from jax.experimental.pallas import tpu as pltpu
from jax.experimental.pallas import tpu_sc as plsc

MAX_LEN = 1000
HIDDEN_DIM = 128

NW = 32
B, N, L, D = 16, 64, 50, 128
PANELS = B * L
PANELS_PER_W = PANELS // NW
L_PER_W = L // 2
CH_SIZES = [3] * 8 + [1]
CH_OFFS = [sum(CH_SIZES[:i]) for i in range(len(CH_SIZES))]
NCH = len(CH_SIZES)
CH_MAX = max(CH_SIZES)
NBUF = 4
GA_SKEW = 1
OUT_SKEW = 2


def _sc_kernel(emb_hbm, pos_hbm, pe_hbm, out_hbm,
               idx_v, pe_sh, b0, b1, b2, b3,
               sem_in, sem_ga, sem_out, sem_idx):
  bufs = (b0, b1, b2, b3)
  sid = lax.axis_index("s")
  wid = sid * 2 + lax.axis_index("c")
  bb = wid // 2
  l0 = (wid % 2) * L_PER_W

  in_d = [None] * NCH
  ga_d = [None] * NCH
  out_d = [None] * NCH

  def start_in(c):
    sz = CH_SIZES[c]
    in_d[c] = pltpu.async_copy(
        emb_hbm.at[bb, pl.ds(l0 + CH_OFFS[c], sz)],
        bufs[c % NBUF].at[pl.ds(0, sz)], sem_in)

  def start_ga(c):
    in_d[c].wait()
    ga_d[c] = [
        pltpu.async_copy(
            pe_sh.at[idx_v.at[CH_OFFS[c] + k]], bufs[c % NBUF].at[k],
            sem_ga, add=True)
        for k in range(CH_SIZES[c])
    ]

  def start_out(c):
    for d in ga_d[c]:
      d.wait()
    sz = CH_SIZES[c]
    out_d[c] = pltpu.async_copy(
        bufs[c % NBUF].at[pl.ds(0, sz)],
        out_hbm.at[bb, pl.ds(l0 + CH_OFFS[c], sz)], sem_out)

  idx_cp = pltpu.async_copy(
      pos_hbm.at[pl.ds(l0, L_PER_W), bb], idx_v, sem_idx)
  for c in range(GA_SKEW):
    start_in(c)

  @pl.when(sid == 0)
  def _stage():
    pltpu.sync_copy(pe_hbm, pe_sh)

  idx_cp.wait()
  plsc.subcore_barrier()

  for c in range(NCH):
    if c >= NBUF:
      out_d[c - NBUF].wait()
    if c >= GA_SKEW:
      start_in(c)
      start_ga(c - GA_SKEW)
    if c >= OUT_SKEW:
      start_out(c - OUT_SKEW)

  for c in range(NCH - GA_SKEW, NCH):
    start_ga(c)
  for c in range(NCH - OUT_SKEW, NCH):
    start_out(c)
  for c in range(max(0, NCH - NBUF), NCH):
    out_d[c].wait()


def kernel(input_emb, position, pe):
  emb_t = jnp.transpose(input_emb, (0, 2, 1, 3))
  pos_t = jnp.transpose(position.astype(jnp.int32), (2, 0, 1))

  run = functools.partial(
      pl.kernel,
      mesh=plsc.VectorSubcoreMesh(core_axis_name="c", subcore_axis_name="s"),
      out_type=jax.ShapeDtypeStruct((B, L, N, D), jnp.float32),
      scratch_types=[
          pltpu.VMEM((PANELS_PER_W, N), jnp.int32),
          pltpu.VMEM_SHARED((MAX_LEN, D), jnp.float32),
          pltpu.VMEM((CH_MAX, N, D), jnp.float32),
          pltpu.VMEM((CH_MAX, N, D), jnp.float32),
          pltpu.VMEM((CH_MAX, N, D), jnp.float32),
          pltpu.VMEM((CH_MAX, N, D), jnp.float32),
          pltpu.SemaphoreType.DMA,
          pltpu.SemaphoreType.DMA,
          pltpu.SemaphoreType.DMA,
          pltpu.SemaphoreType.DMA,
      ],
  )(_sc_kernel)

  out_t = run(emb_t, pos_t, pe)
  return jnp.transpose(out_t, (0, 2, 1, 3))


# speedup vs baseline: 1.0182x; 1.0037x over previous
import functools

import jax
import jax.numpy as jnp
from jax import lax
from jax.experimental import pallas as pl
from jax.experimental.pallas import tpu as pltpu
from jax.experimental.pallas import tpu_sc as plsc

MAX_LEN = 1000
HIDDEN_DIM = 128

NW = 32
B, N, L, D = 16, 64, 50, 128
PANELS = B * L
PANELS_PER_W = PANELS // NW
L_PER_W = L // 2
CH_SIZES = [3] * 8 + [1]
CH_OFFS = [sum(CH_SIZES[:i]) for i in range(len(CH_SIZES))]
NCH = len(CH_SIZES)
CH_MAX = max(CH_SIZES)
NBUF = 4
GA_SKEW = 2
OUT_SKEW = 3


def _sc_kernel(emb_hbm, pos_hbm, pe_hbm, out_hbm,
               idx_v, pe_sh, b0, b1, b2, b3,
               sem_in, sem_ga, sem_out, sem_idx):
  bufs = (b0, b1, b2, b3)
  sid = lax.axis_index("s")
  wid = sid * 2 + lax.axis_index("c")
  bb = wid // 2
  l0 = (wid % 2) * L_PER_W

  in_d = [None] * NCH
  ga_d = [None] * NCH
  out_d = [None] * NCH

  def start_in(c):
    sz = CH_SIZES[c]
    in_d[c] = pltpu.async_copy(
        emb_hbm.at[bb, pl.ds(l0 + CH_OFFS[c], sz)],
        bufs[c % NBUF].at[pl.ds(0, sz)], sem_in)

  def start_ga(c):
    in_d[c].wait()
    ga_d[c] = [
        pltpu.async_copy(
            pe_sh.at[idx_v.at[CH_OFFS[c] + k]], bufs[c % NBUF].at[k],
            sem_ga, add=True)
        for k in range(CH_SIZES[c])
    ]

  def start_out(c):
    for d in ga_d[c]:
      d.wait()
    sz = CH_SIZES[c]
    out_d[c] = pltpu.async_copy(
        bufs[c % NBUF].at[pl.ds(0, sz)],
        out_hbm.at[bb, pl.ds(l0 + CH_OFFS[c], sz)], sem_out)

  idx_cp = pltpu.async_copy(
      pos_hbm.at[pl.ds(l0, L_PER_W), bb], idx_v, sem_idx)
  for c in range(GA_SKEW):
    start_in(c)

  @pl.when(sid == 0)
  def _stage():
    pltpu.sync_copy(pe_hbm, pe_sh)

  idx_cp.wait()
  plsc.subcore_barrier()

  for c in range(NCH):
    if c >= NBUF:
      out_d[c - NBUF].wait()
    if c >= GA_SKEW:
      start_in(c)
      start_ga(c - GA_SKEW)
    if c >= OUT_SKEW:
      start_out(c - OUT_SKEW)

  for c in range(NCH - GA_SKEW, NCH):
    start_ga(c)
  for c in range(NCH - OUT_SKEW, NCH):
    start_out(c)
  for c in range(max(0, NCH - NBUF), NCH):
    out_d[c].wait()


def kernel(input_emb, position, pe):
  emb_t = jnp.transpose(input_emb, (0, 2, 1, 3))
  pos_t = jnp.transpose(position.astype(jnp.int32), (2, 0, 1))

  run = functools.partial(
      pl.kernel,
      mesh=plsc.VectorSubcoreMesh(core_axis_name="c", subcore_axis_name="s"),
      out_type=jax.ShapeDtypeStruct((B, L, N, D), jnp.float32),
      scratch_types=[
          pltpu.VMEM((PANELS_PER_W, N), jnp.int32),
          pltpu.VMEM_SHARED((MAX_LEN, D), jnp.float32),
          pltpu.VMEM((CH_MAX, N, D), jnp.float32),
          pltpu.VMEM((CH_MAX, N, D), jnp.float32),
          pltpu.VMEM((CH_MAX, N, D), jnp.float32),
          pltpu.VMEM((CH_MAX, N, D), jnp.float32),
          pltpu.SemaphoreType.DMA,
          pltpu.SemaphoreType.DMA,
          pltpu.SemaphoreType.DMA,
          pltpu.SemaphoreType.DMA,
      ],
  )(_sc_kernel)

  out_t = run(emb_t, pos_t, pe)
  return jnp.transpose(out_t, (0, 2, 1, 3))
